# Initial kernel scaffold; baseline (speedup 1.0000x reference)
import jax
import jax.numpy as jnp
from jax.experimental import pallas as pl


def kernel(x, edge_index, W1_0, b1_0, W2_0, b2_0, W1_1, b1_1, W2_1, b2_1):
    raise NotImplementedError("write your pallas kernel here")


# speedup vs baseline: 4.4238x; 4.4238x over previous
import functools

import jax
import jax.numpy as jnp
from jax import lax
from jax.experimental import pallas as pl
from jax.experimental.pallas import tpu as pltpu
from jax.experimental.pallas import tpu_sc as plsc

N = 10000
D = 128
E = 320000

NC = 2
NS = 16
ROW_STEP = 624
EDGES_PER_WORKER = E // (NC * NS)
CHUNK = 80
NCHUNK = EDGES_PER_WORKER // CHUNK
ZR = 16


def _sc_scatter_partials(feat, src, dst):
    mesh = plsc.VectorSubcoreMesh(core_axis_name="c", subcore_axis_name="s")

    @functools.partial(
        pl.kernel,
        out_type=jax.ShapeDtypeStruct((NC * N, D), jnp.float32),
        mesh=mesh,
        scratch_types=[
            pltpu.VMEM_SHARED((N, D), jnp.float32),
            pltpu.VMEM((ZR, D), jnp.float32),
            pltpu.VMEM((CHUNK,), jnp.int32),
            pltpu.VMEM((CHUNK,), jnp.int32),
            pltpu.VMEM((CHUNK, D), jnp.float32),
            pltpu.SemaphoreType.DMA,
        ],
    )
    def k(feat_hbm, src_hbm, dst_hbm, out_hbm, acc, zbuf, src_v, dst_v, rows_v, sem):
        c = lax.axis_index("c")
        s = lax.axis_index("s")
        row0 = s * ROW_STEP
        ngran = jnp.where(s == NS - 1, 640 // ZR, ROW_STEP // ZR)

        zv = jnp.zeros((16,), jnp.float32)

        def zb(i, carry):
            zbuf[i // (D // 16), pl.ds((i % (D // 16)) * 16, 16)] = zv
            return carry

        lax.fori_loop(0, ZR * (D // 16), zb, 0)

        def ib(j, carry):
            pltpu.sync_copy(zbuf, acc.at[pl.ds(row0 + j * ZR, ZR)])
            return carry

        lax.fori_loop(0, ngran, ib, 0)
        plsc.subcore_barrier()

        base = (c * NS + s) * EDGES_PER_WORKER

        def body(i, carry):
            off = base + i * CHUNK
            pltpu.sync_copy(src_hbm.at[pl.ds(off, CHUNK)], src_v)
            pltpu.sync_copy(dst_hbm.at[pl.ds(off, CHUNK)], dst_v)
            pltpu.async_copy(feat_hbm.at[src_v], rows_v, sem).wait()
            pltpu.sync_copy(rows_v, acc.at[dst_v], add=True)
            return carry

        lax.fori_loop(0, NCHUNK, body, 0)
        plsc.subcore_barrier()

        def wb(j, carry):
            pltpu.sync_copy(acc.at[pl.ds(row0 + j * ZR, ZR)],
                            out_hbm.at[pl.ds(c * N + row0 + j * ZR, ZR)])
            return carry

        lax.fori_loop(0, ngran, wb, 0)

    return k(feat, src, dst)


def _mlp(xin, partials, W1, b1, W2, b2, final_relu):
    R = 1000
    nblk = N // R

    def body(x_ref, p0_ref, p1_ref, w1_ref, b1_ref, w2_ref, b2_ref, o_ref):
        h = x_ref[...] + p0_ref[...] + p1_ref[...]
        h = jnp.dot(h, w1_ref[...], preferred_element_type=jnp.float32) + b1_ref[...]
        h = jnp.maximum(h, 0.0)
        o = jnp.dot(h, w2_ref[...], preferred_element_type=jnp.float32) + b2_ref[...]
        if final_relu:
            o = jnp.maximum(o, 0.0)
        o_ref[...] = o

    return pl.pallas_call(
        body,
        grid=(nblk,),
        in_specs=[
            pl.BlockSpec((R, D), lambda i: (i, 0)),
            pl.BlockSpec((R, D), lambda i: (i, 0)),
            pl.BlockSpec((R, D), lambda i: (i + nblk, 0)),
            pl.BlockSpec((D, D), lambda i: (0, 0)),
            pl.BlockSpec((1, D), lambda i: (0, 0)),
            pl.BlockSpec((D, D), lambda i: (0, 0)),
            pl.BlockSpec((1, D), lambda i: (0, 0)),
        ],
        out_specs=pl.BlockSpec((R, D), lambda i: (i, 0)),
        out_shape=jax.ShapeDtypeStruct((N, D), jnp.float32),
    )(xin, partials, partials, W1, b1.reshape(1, D), W2, b2.reshape(1, D))


def kernel(x, edge_index, W1_0, b1_0, W2_0, b2_0, W1_1, b1_1, W2_1, b2_1):
    src = edge_index[0].astype(jnp.int32)
    dst = edge_index[1].astype(jnp.int32)
    p = _sc_scatter_partials(x, src, dst)
    h = _mlp(x, p, W1_0, b1_0, W2_0, b2_0, final_relu=True)
    p = _sc_scatter_partials(h, src, dst)
    return _mlp(h, p, W1_1, b1_1, W2_1, b2_1, final_relu=False)
